# Initial kernel scaffold; baseline (speedup 1.0000x reference)
import jax
import jax.numpy as jnp
from jax.experimental import pallas as pl


def kernel(X, H, edge_index, edge_feature, wmx_x, wme_x, wsk_x, bsk_x, wmx_h, wme_h, wsk_h, bsk_h):
    raise NotImplementedError("write your pallas kernel here")


# speedup vs baseline: 3.4011x; 3.4011x over previous
import functools

import jax
import jax.numpy as jnp
from jax.experimental import pallas as pl
from jax.experimental.pallas import tpu as pltpu

_F32 = jnp.float32


def _round_up(a, m):
    return (a + m - 1) // m * m


def _scatter_xh_kernel(src_ref, dst_ref, xh_ref, ef_ref, macc_ref, *, eh, unroll):
    c = pl.program_id(0)
    macc_ref[...] = jnp.zeros_like(macc_ref)
    e0 = c * eh

    def chunk(k, carry):
        b = k * unroll
        for j in range(unroll):
            li = b + j
            s = src_ref[e0 + li]
            d = dst_ref[e0 + li]
            g = xh_ref[s]
            fe = ef_ref[li]
            macc_ref[d] = macc_ref[d] + jnp.concatenate([g, fe], axis=1)
        return carry

    jax.lax.fori_loop(0, eh // unroll, chunk, 0)


def _gates_scatter_kernel(src_ref, dst_ref, macc_ref, x_ref, h_ref,
                          wt_ref, wskc_ref, b4_ref,
                          z_ref, hb_ref, cacc_ref, hr_ref,
                          *, eh, unroll, cc, rn, d2):
    c = pl.program_id(0)
    M = macc_ref[0:rn, :] + macc_ref[rn:2 * rn, :]
    cin = x_ref.shape[1]
    T = (jnp.dot(M, wt_ref[0:d2, :], preferred_element_type=_F32)
         + jnp.dot(x_ref[...], wt_ref[d2:d2 + cin, :], preferred_element_type=_F32)
         + jnp.dot(h_ref[...], wt_ref[d2 + cin:, :], preferred_element_type=_F32)
         + b4_ref[...])
    Z = jax.nn.sigmoid(T[:, 0:cc])
    R = jax.nn.sigmoid(T[:, cc:2 * cc])
    HR = h_ref[...] * R
    hb = (T[:, 2 * cc:3 * cc] + T[:, 3 * cc:4 * cc]
          + jnp.dot(HR, wskc_ref[...], preferred_element_type=_F32))
    z_ref[0] = Z
    hb_ref[0] = hb
    hr_ref[...] = HR.reshape(rn, 1, cc)
    cacc_ref[...] = jnp.zeros_like(cacc_ref)
    e0 = c * eh

    def chunk(k, carry):
        b = k * unroll
        for j in range(unroll):
            gi = e0 + b + j
            s = src_ref[gi]
            d = dst_ref[gi]
            cacc_ref[d] = cacc_ref[d] + hr_ref[s]
        return carry

    jax.lax.fori_loop(0, eh // unroll, chunk, 0)


def _finish_kernel(z_ref, hb_ref, h_ref, cacc_ref, wmc_ref, out_ref):
    cs = cacc_ref[0] + cacc_ref[1]
    ht = jnp.tanh(hb_ref[...]
                  + jnp.dot(cs, wmc_ref[...], preferred_element_type=_F32))
    z = z_ref[...]
    out_ref[...] = z * h_ref[...] + (1.0 - z) * ht


def kernel(X, H, edge_index, edge_feature,
           wmx_x, wme_x, wsk_x, bsk_x, wmx_h, wme_h, wsk_h, bsk_h):
    N, c_in = X.shape
    C = H.shape[1]
    E, De = edge_feature.shape
    n_cores = 2
    unroll = 8

    eh = _round_up((E + n_cores - 1) // n_cores, unroll)
    e_pad = n_cores * eh
    rn = _round_up(N, 16) + 16
    rh = rn // 2
    dummy = rn - 1
    d1 = c_in + C
    d2 = d1 + De

    src = jnp.zeros((e_pad,), jnp.int32).at[:E].set(edge_index[0].astype(jnp.int32))
    dst = jnp.full((e_pad,), dummy, jnp.int32).at[:E].set(edge_index[1].astype(jnp.int32))

    Xf = X.astype(_F32)
    Hf = H.astype(_F32)
    XH = (jnp.zeros((rn, 1, d1), _F32)
          .at[:N, 0, :c_in].set(Xf)
          .at[:N, 0, c_in:].set(Hf))
    EF3 = jnp.zeros((e_pad, 1, De), _F32).at[:E, 0, :].set(edge_feature.astype(_F32))
    X_p = jnp.zeros((rn, c_in), _F32).at[:N].set(Xf)
    H_p = jnp.zeros((rn, C), _F32).at[:N].set(Hf)

    z_xc = jnp.zeros((c_in, C), _F32)
    z_cc = jnp.zeros((C, C), _F32)
    rows_sx = jnp.concatenate([wmx_x[0], wmx_x[1], wmx_x[2], z_xc], axis=1)
    rows_sh = jnp.concatenate([wmx_h[0], wmx_h[1], z_cc, z_cc], axis=1)
    rows_se = jnp.concatenate([wme_x[0] + wme_h[0], wme_x[1] + wme_h[1],
                               wme_x[2], wme_h[2]], axis=1)
    rows_x = jnp.concatenate([wsk_x[0], wsk_x[1], wsk_x[2], z_xc], axis=1)
    rows_h = jnp.concatenate([wsk_h[0], wsk_h[1], z_cc, z_cc], axis=1)
    WT = jnp.concatenate([rows_sx, rows_sh, rows_se, rows_x, rows_h],
                         axis=0).astype(_F32)
    b4 = jnp.concatenate([bsk_x[0] + bsk_h[0], bsk_x[1] + bsk_h[1],
                          bsk_x[2], bsk_h[2]], axis=1).astype(_F32)
    wskc = wsk_h[2].astype(_F32)
    wmc = wmx_h[2].astype(_F32)

    k1_spec = pltpu.PrefetchScalarGridSpec(
        num_scalar_prefetch=2,
        grid=(n_cores,),
        in_specs=[
            pl.BlockSpec((rn, 1, d1), lambda c, *_: (0, 0, 0)),
            pl.BlockSpec((eh, 1, De), lambda c, *_: (c, 0, 0)),
        ],
        out_specs=pl.BlockSpec((rn, 1, d2), lambda c, *_: (c, 0, 0)),
    )
    macc = pl.pallas_call(
        functools.partial(_scatter_xh_kernel, eh=eh, unroll=unroll),
        out_shape=jax.ShapeDtypeStruct((n_cores * rn, 1, d2), _F32),
        grid_spec=k1_spec,
        compiler_params=pltpu.CompilerParams(
            dimension_semantics=("parallel",),
            vmem_limit_bytes=50 * 1024 * 1024,
        ),
    )(src, dst, XH, EF3)

    macc2 = macc.reshape(n_cores * rn, d2)

    k2_spec = pltpu.PrefetchScalarGridSpec(
        num_scalar_prefetch=2,
        grid=(n_cores,),
        in_specs=[
            pl.BlockSpec((n_cores * rn, d2), lambda c, *_: (0, 0)),
            pl.BlockSpec((rn, c_in), lambda c, *_: (0, 0)),
            pl.BlockSpec((rn, C), lambda c, *_: (0, 0)),
            pl.BlockSpec((d2 + c_in + C, 4 * C), lambda c, *_: (0, 0)),
            pl.BlockSpec((C, C), lambda c, *_: (0, 0)),
            pl.BlockSpec((1, 4 * C), lambda c, *_: (0, 0)),
        ],
        out_specs=[
            pl.BlockSpec((1, rn, C), lambda c, *_: (c, 0, 0)),
            pl.BlockSpec((1, rn, C), lambda c, *_: (c, 0, 0)),
            pl.BlockSpec((rn, 1, C), lambda c, *_: (c, 0, 0)),
        ],
        scratch_shapes=[pltpu.VMEM((rn, 1, C), _F32)],
    )
    z2, hb2, cacc = pl.pallas_call(
        functools.partial(_gates_scatter_kernel, eh=eh, unroll=unroll,
                          cc=C, rn=rn, d2=d2),
        out_shape=[
            jax.ShapeDtypeStruct((n_cores, rn, C), _F32),
            jax.ShapeDtypeStruct((n_cores, rn, C), _F32),
            jax.ShapeDtypeStruct((n_cores * rn, 1, C), _F32),
        ],
        grid_spec=k2_spec,
        compiler_params=pltpu.CompilerParams(
            dimension_semantics=("parallel",),
            vmem_limit_bytes=60 * 1024 * 1024,
        ),
    )(src, dst, macc2, X_p, H_p, WT, wskc, b4)

    cacc3 = cacc.reshape(n_cores, rn, C)

    out = pl.pallas_call(
        _finish_kernel,
        grid=(n_cores,),
        in_specs=[
            pl.BlockSpec((rh, C), lambda c: (c, 0)),
            pl.BlockSpec((rh, C), lambda c: (c, 0)),
            pl.BlockSpec((rh, C), lambda c: (c, 0)),
            pl.BlockSpec((n_cores, rh, C), lambda c: (0, c, 0)),
            pl.BlockSpec((C, C), lambda c: (0, 0)),
        ],
        out_specs=pl.BlockSpec((rh, C), lambda c: (c, 0)),
        out_shape=jax.ShapeDtypeStruct((rn, C), _F32),
        compiler_params=pltpu.CompilerParams(
            dimension_semantics=("parallel",),
        ),
    )(z2[0], hb2[0], H_p, cacc3, wmc)

    return out[:N]
